# Initial kernel scaffold; baseline (speedup 1.0000x reference)
import jax
import jax.numpy as jnp
from jax.experimental import pallas as pl


def kernel(x, enc_w1, enc_b1, enc_w2, enc_b2, cb_w, dec_w1, dec_b1, dec_w2, dec_b2, kp_w1, kp_b1, kp_w2, kp_b2, kp_w3, kp_b3, k_scale):
    raise NotImplementedError("write your pallas kernel here")


# speedup vs baseline: 13.9240x; 13.9240x over previous
import jax
import jax.numpy as jnp
from jax.experimental import pallas as pl

_QDIM = 512
_ROWS = 1024


def _fused_kernel(x_ref, ew1_ref, eb1_ref, ew2_ref, eb2_ref, cbw_ref,
                  dw1_ref, db1_ref, dw2_ref, db2_ref,
                  kw1x_ref, kw1l_ref, kb1_ref, kw2_ref, kb2_ref,
                  kw3_ref, kb3s_ref, ks2_ref, ltri_ref,
                  recon_ref, khot_ref, k_ref):
    f32 = jnp.float32
    x = x_ref[...]

    h = jnp.maximum(
        jnp.dot(x, ew1_ref[...], preferred_element_type=f32) + eb1_ref[...], 0.0)
    logits = jnp.dot(h, ew2_ref[...], preferred_element_type=f32) + eb2_ref[...]

    h1 = jnp.maximum(
        jnp.dot(x, kw1x_ref[...], preferred_element_type=f32)
        + jnp.dot(logits, kw1l_ref[...], preferred_element_type=f32)
        + kb1_ref[...], 0.0)
    h2 = jnp.maximum(
        jnp.dot(h1, kw2_ref[...], preferred_element_type=f32) + kb2_ref[...], 0.0)
    z = jnp.sum(h2 * kw3_ref[...], axis=-1, keepdims=True) + kb3s_ref[0, 0]
    k = jax.nn.sigmoid(z) * float(_QDIM)
    k = jnp.clip(k * ks2_ref[0, 0], 1.0, float(_QDIM))
    n = jnp.ceil(k)

    b = jax.lax.bitcast_convert_type(logits, jnp.int32)
    m = jnp.where(b >= 0, b, b ^ jnp.int32(0x7FFFFFFF))

    def _cnt(trial):
        return jnp.sum((m >= trial).astype(f32), axis=-1, keepdims=True)

    zero = jnp.zeros_like(n, dtype=jnp.int32)
    thr = jnp.where(_cnt(zero) >= n, zero, jnp.full_like(zero, jnp.int32(-2**31)))
    for bit in range(30, -1, -1):
        trial = thr | jnp.int32(1 << bit)
        thr = jnp.where(_cnt(trial) >= n, trial, thr)

    gt = (m > thr).astype(f32)
    eq = (m == thr).astype(f32)
    extra = n - jnp.sum(gt, axis=-1, keepdims=True)
    prefix = jnp.dot(eq, ltri_ref[...], preferred_element_type=f32)
    khot = gt + eq * (prefix < extra).astype(f32)
    khot_ref[...] = khot

    q = jnp.dot(khot, cbw_ref[...], preferred_element_type=f32) / k
    d1 = jnp.maximum(
        jnp.dot(q, dw1_ref[...], preferred_element_type=f32) + db1_ref[...], 0.0)
    recon_ref[...] = jnp.dot(d1, dw2_ref[...], preferred_element_type=f32) + db2_ref[...]
    k_ref[...] = k


def kernel(x, enc_w1, enc_b1, enc_w2, enc_b2, cb_w, dec_w1, dec_b1, dec_w2,
           dec_b2, kp_w1, kp_b1, kp_w2, kp_b2, kp_w3, kp_b3, k_scale):
    batch, in_dim = x.shape
    n_hdim = enc_w1.shape[1]
    grid = (batch // _ROWS,)

    kw1x = kp_w1[:in_dim]
    kw1l = kp_w1[in_dim:]
    ks2 = (jax.nn.sigmoid(k_scale) * 2.0).reshape(1, 1)
    ltri = (jnp.arange(_QDIM)[:, None] < jnp.arange(_QDIM)[None, :]).astype(x.dtype)

    def row_blk(shape):
        return pl.BlockSpec(shape, lambda i: (i, 0))

    def rep_blk(shape):
        return pl.BlockSpec(shape, lambda i: (0, 0))

    recon, khot, k = pl.pallas_call(
        _fused_kernel,
        grid=grid,
        in_specs=[
            row_blk((_ROWS, in_dim)),
            rep_blk((in_dim, n_hdim)), rep_blk((1, n_hdim)),
            rep_blk((n_hdim, _QDIM)), rep_blk((1, _QDIM)),
            rep_blk((_QDIM, cb_w.shape[1])),
            rep_blk((cb_w.shape[1], n_hdim)), rep_blk((1, n_hdim)),
            rep_blk((n_hdim, in_dim)), rep_blk((1, in_dim)),
            rep_blk((in_dim, n_hdim)), rep_blk((_QDIM, n_hdim)),
            rep_blk((1, n_hdim)),
            rep_blk((n_hdim, n_hdim)), rep_blk((1, n_hdim)),
            rep_blk((1, n_hdim)), rep_blk((1, 1)), rep_blk((1, 1)),
            rep_blk((_QDIM, _QDIM)),
        ],
        out_specs=[
            row_blk((_ROWS, in_dim)),
            row_blk((_ROWS, _QDIM)),
            row_blk((_ROWS, 1)),
        ],
        out_shape=[
            jax.ShapeDtypeStruct((batch, in_dim), x.dtype),
            jax.ShapeDtypeStruct((batch, _QDIM), x.dtype),
            jax.ShapeDtypeStruct((batch, 1), x.dtype),
        ],
    )(
        x,
        enc_w1, enc_b1.reshape(1, -1), enc_w2, enc_b2.reshape(1, -1), cb_w,
        dec_w1, dec_b1.reshape(1, -1), dec_w2, dec_b2.reshape(1, -1),
        kw1x, kw1l, kp_b1.reshape(1, -1), kp_w2, kp_b2.reshape(1, -1),
        kp_w3.reshape(1, -1), kp_b3.reshape(1, 1), ks2, ltri,
    )
    return (recon, khot, 0.0, k)
